# baseline (speedup 1.0000x reference)
import functools

import jax
import jax.numpy as jnp
from jax import lax
from jax.experimental import pallas as pl
from jax.experimental.pallas import tpu as pltpu
from jax.experimental.pallas import tpu_sc as plsc

N = 10000
E = 320000
N_GRAPHS = 64

NC = 2
NS = 16
NW = NC * NS
CHUNK = 128
NCHUNK = 80
E_PAD = NW * NCHUNK * CHUNK
STRIPE = 640
NP = NS * STRIPE


def _make_sc_agg(F):
    mesh = plsc.VectorSubcoreMesh(core_axis_name="c", subcore_axis_name="s")

    @functools.partial(
        pl.kernel, mesh=mesh,
        compiler_params=pltpu.CompilerParams(use_tc_tiling_on_sc=False),
        out_type=jax.ShapeDtypeStruct((NW, STRIPE, F), jnp.float32),
        scratch_types=[
            pltpu.VMEM((NCHUNK, CHUNK), jnp.int32),
            pltpu.VMEM((NCHUNK, CHUNK), jnp.int32),
            [pltpu.VMEM((CHUNK, F), jnp.float32)] * 4,
            pltpu.VMEM((64, F), jnp.float32),
            pltpu.VMEM_SHARED((NP, F), jnp.float32),
            [pltpu.SemaphoreType.DMA] * 4,
        ],
    )
    def k(g_hbm, srcR, dstR, out_hbm, src_v, dst_v, bufs, zbuf, acc, sems):
        c = lax.axis_index("c")
        s = lax.axis_index("s")
        wid = c * NS + s

        for i in range(64):
            for j in range(F // 16):
                zbuf[i, 16 * j:16 * (j + 1)] = jnp.zeros((16,), jnp.float32)

        def zstripe(kk, carry):
            pltpu.sync_copy(zbuf, acc.at[pl.ds(s * STRIPE + kk * 64, 64)])
            return carry
        lax.fori_loop(0, STRIPE // 64, zstripe, 0)
        plsc.subcore_barrier()

        pltpu.sync_copy(srcR.at[wid], src_v)
        pltpu.sync_copy(dstR.at[wid], dst_v)

        def gath(j, b):
            pltpu.async_copy(g_hbm.at[src_v.at[j]], bufs[b], sems[b])

        def wait(j, b):
            pltpu.make_async_copy(g_hbm.at[src_v.at[j]], bufs[b],
                                  sems[b]).wait()

        def scat(j, b):
            pltpu.sync_copy(bufs[b], acc.at[dst_v.at[j]], add=True)

        for b in range(4):
            gath(b, b)

        def quad(t, carry):
            j0 = 4 * t
            for b in range(4):
                wait(j0 + b, b)
                scat(j0 + b, b)
                gath(j0 + b + 4, b)
            return carry
        lax.fori_loop(0, NCHUNK // 4 - 1, quad, 0)
        for b in range(4):
            j = NCHUNK - 4 + b
            wait(j, b)
            scat(j, b)

        plsc.subcore_barrier()
        pltpu.sync_copy(acc.at[pl.ds(s * STRIPE, STRIPE)], out_hbm.at[wid])

    return k


def _make_sc_count():
    F = 16
    mesh = plsc.VectorSubcoreMesh(core_axis_name="c", subcore_axis_name="s")

    @functools.partial(
        pl.kernel, mesh=mesh,
        compiler_params=pltpu.CompilerParams(use_tc_tiling_on_sc=False),
        out_type=jax.ShapeDtypeStruct((NW, STRIPE, F), jnp.float32),
        scratch_types=[
            pltpu.VMEM((NCHUNK, CHUNK), jnp.int32),
            pltpu.VMEM((CHUNK, F), jnp.float32),
            pltpu.VMEM((64, F), jnp.float32),
            pltpu.VMEM_SHARED((NP, F), jnp.float32),
        ],
    )
    def k(dstR, out_hbm, dst_v, ones, zbuf, acc):
        c = lax.axis_index("c")
        s = lax.axis_index("s")
        wid = c * NS + s

        for i in range(64):
            zbuf[i, 0:16] = jnp.zeros((16,), jnp.float32)
        for i in range(CHUNK):
            ones[i, 0:16] = jnp.ones((16,), jnp.float32)

        def zstripe(kk, carry):
            pltpu.sync_copy(zbuf, acc.at[pl.ds(s * STRIPE + kk * 64, 64)])
            return carry
        lax.fori_loop(0, STRIPE // 64, zstripe, 0)
        plsc.subcore_barrier()

        pltpu.sync_copy(dstR.at[wid], dst_v)

        def chunk(j, carry):
            pltpu.sync_copy(ones, acc.at[dst_v.at[j]], add=True)
            return carry
        lax.fori_loop(0, NCHUNK, chunk, 0)

        plsc.subcore_barrier()
        pltpu.sync_copy(acc.at[pl.ds(s * STRIPE, STRIPE)], out_hbm.at[wid])

    return k


_sc_count = _make_sc_count()


_sc_agg = {f: _make_sc_agg(f) for f in (16, 32, 64)}


def _mm_body(x_ref, w_ref, o_ref):
    o_ref[...] = jnp.dot(x_ref[...], w_ref[...],
                         preferred_element_type=jnp.float32)


def _pallas_matmul(x, w):
    m, k = x.shape
    _, n = w.shape
    bm = 512
    grid = (pl.cdiv(m, bm),)
    return pl.pallas_call(
        _mm_body,
        grid=grid,
        in_specs=[
            pl.BlockSpec((bm, k), lambda i: (i, 0)),
            pl.BlockSpec((k, n), lambda i: (0, 0)),
        ],
        out_specs=pl.BlockSpec((bm, n), lambda i: (i, 0)),
        out_shape=jax.ShapeDtypeStruct((m, n), jnp.float32),
    )(x, w)


def _agg(g, srcR, dstR):
    p = _sc_agg[g.shape[1]](g, srcR, dstR).reshape(NC, NP, g.shape[1])
    return p[0, :N] + p[1, :N]


def kernel(x, edge_index, batch, W1, b1, W2, b2, Wc0, bc0, Wc1, bc1,
           Wf1, bf1, Wf2, bf2, Wf3, bf3):
    src, dst = edge_index[0], edge_index[1]
    act = jax.nn.elu

    pad = E_PAD - E
    srcR = jnp.concatenate([src, jnp.zeros((pad,), jnp.int32)]) \
        .reshape(NW, NCHUNK, CHUNK)
    dstR = jnp.concatenate([dst, jnp.full((pad,), N, jnp.int32)]) \
        .reshape(NW, NCHUNK, CHUNK)

    cnt = _sc_count(dstR).reshape(NC, NP, 16)
    deg = cnt[0, :N, 0] + cnt[1, :N, 0] + 1.0
    dinv = jax.lax.rsqrt(deg)

    def conv(h, b):
        g = dinv[:, None] * h
        return dinv[:, None] * (_agg(g, srcR, dstR) + g) + b

    x1 = act(conv(_pallas_matmul(x, W1), b1))
    x3 = act(conv(_pallas_matmul(x1, W2), b2))
    x3 = act(conv(_pallas_matmul(x3, Wc0), bc0))
    x3 = act(conv(_pallas_matmul(x3, Wc1), bc1))

    pooled = jax.ops.segment_max(x3, batch, num_segments=N_GRAPHS,
                                 indices_are_sorted=True)
    pooled = jnp.where(jnp.isfinite(pooled), pooled, 0.0)
    h = act(pooled @ Wf1 + bf1)
    h = act(h @ Wf2 + bf2)
    logits = h @ Wf3 + bf3
    return jax.nn.log_softmax(logits, axis=1)


# speedup vs baseline: 1.9375x; 1.9375x over previous
import functools

import jax
import jax.numpy as jnp
from jax import lax
from jax.experimental import pallas as pl
from jax.experimental.pallas import tpu as pltpu
from jax.experimental.pallas import tpu_sc as plsc

N = 10000
E = 320000
N_GRAPHS = 64

NC = 2
NS = 16
NW = NC * NS
CHUNK = 128
NCHUNK = 80
E_PAD = NW * NCHUNK * CHUNK
STRIPE = 640
NP = NS * STRIPE


NCHUNK_T = E_PAD // (NS * CHUNK)


def _make_sc_agg(F):
    Fh = F // 2
    mesh = plsc.VectorSubcoreMesh(core_axis_name="c", subcore_axis_name="s")

    @functools.partial(
        pl.kernel, mesh=mesh,
        compiler_params=pltpu.CompilerParams(use_tc_tiling_on_sc=False),
        out_type=jax.ShapeDtypeStruct((NC, NS, STRIPE, Fh), jnp.float32),
        scratch_types=[
            pltpu.VMEM((NCHUNK_T, CHUNK), jnp.int32),
            pltpu.VMEM((NCHUNK_T, CHUNK), jnp.int32),
            [pltpu.VMEM((CHUNK, Fh), jnp.float32)] * 4,
            pltpu.VMEM((64, Fh), jnp.float32),
            pltpu.VMEM_SHARED((NP, Fh), jnp.float32),
            pltpu.VMEM_SHARED((N, Fh), jnp.float32),
            [pltpu.SemaphoreType.DMA] * 4,
        ],
    )
    def k(g_hbm, srcR, dstR, out_hbm, src_v, dst_v, bufs, zbuf, acc, g_sp,
          sems):
        c = lax.axis_index("c")
        s = lax.axis_index("s")

        for i in range(64):
            for j in range(Fh // 16):
                zbuf[i, 16 * j:16 * (j + 1)] = jnp.zeros((16,), jnp.float32)

        def zstripe(kk, carry):
            pltpu.sync_copy(zbuf, acc.at[pl.ds(s * STRIPE + kk * 64, 64)])
            return carry
        lax.fori_loop(0, STRIPE // 64, zstripe, 0)
        pltpu.sync_copy(g_hbm.at[c].at[pl.ds(s * (N // NS), N // NS)],
                        g_sp.at[pl.ds(s * (N // NS), N // NS)])
        plsc.subcore_barrier()

        pltpu.sync_copy(srcR.at[s], src_v)
        pltpu.sync_copy(dstR.at[s], dst_v)

        def gath(j, b):
            pltpu.async_copy(g_sp.at[src_v.at[j]], bufs[b], sems[b])

        def wait(j, b):
            pltpu.make_async_copy(g_sp.at[src_v.at[j]], bufs[b],
                                  sems[b]).wait()

        def scat(j, b):
            pltpu.sync_copy(bufs[b], acc.at[dst_v.at[j]], add=True)

        for b in range(4):
            gath(b, b)

        def quad(t, carry):
            j0 = 4 * t
            for b in range(4):
                wait(j0 + b, b)
                scat(j0 + b, b)
                gath(j0 + b + 4, b)
            return carry
        lax.fori_loop(0, NCHUNK_T // 4 - 1, quad, 0)
        for b in range(4):
            j = NCHUNK_T - 4 + b
            wait(j, b)
            scat(j, b)

        plsc.subcore_barrier()
        pltpu.sync_copy(acc.at[pl.ds(s * STRIPE, STRIPE)], out_hbm.at[c, s])

    return k


def _make_sc_count():
    F = 16
    mesh = plsc.VectorSubcoreMesh(core_axis_name="c", subcore_axis_name="s")

    @functools.partial(
        pl.kernel, mesh=mesh,
        compiler_params=pltpu.CompilerParams(use_tc_tiling_on_sc=False),
        out_type=jax.ShapeDtypeStruct((NW, STRIPE, F), jnp.float32),
        scratch_types=[
            pltpu.VMEM((NCHUNK, CHUNK), jnp.int32),
            pltpu.VMEM((CHUNK, F), jnp.float32),
            pltpu.VMEM((64, F), jnp.float32),
            pltpu.VMEM_SHARED((NP, F), jnp.float32),
        ],
    )
    def k(dstR, out_hbm, dst_v, ones, zbuf, acc):
        c = lax.axis_index("c")
        s = lax.axis_index("s")
        wid = c * NS + s

        for i in range(64):
            zbuf[i, 0:16] = jnp.zeros((16,), jnp.float32)
        for i in range(CHUNK):
            ones[i, 0:16] = jnp.ones((16,), jnp.float32)

        def zstripe(kk, carry):
            pltpu.sync_copy(zbuf, acc.at[pl.ds(s * STRIPE + kk * 64, 64)])
            return carry
        lax.fori_loop(0, STRIPE // 64, zstripe, 0)
        plsc.subcore_barrier()

        pltpu.sync_copy(dstR.at[wid], dst_v)

        def chunk(j, carry):
            pltpu.sync_copy(ones, acc.at[dst_v.at[j]], add=True)
            return carry
        lax.fori_loop(0, NCHUNK, chunk, 0)

        plsc.subcore_barrier()
        pltpu.sync_copy(acc.at[pl.ds(s * STRIPE, STRIPE)], out_hbm.at[wid])

    return k


_sc_count = _make_sc_count()


_sc_agg = {f: _make_sc_agg(f) for f in (32, 64)}


def _mm_body(x_ref, w_ref, o_ref):
    o_ref[...] = jnp.dot(x_ref[...], w_ref[...],
                         preferred_element_type=jnp.float32)


def _pallas_matmul(x, w):
    m, k = x.shape
    _, n = w.shape
    bm = 512
    grid = (pl.cdiv(m, bm),)
    return pl.pallas_call(
        _mm_body,
        grid=grid,
        in_specs=[
            pl.BlockSpec((bm, k), lambda i: (i, 0)),
            pl.BlockSpec((k, n), lambda i: (0, 0)),
        ],
        out_specs=pl.BlockSpec((bm, n), lambda i: (i, 0)),
        out_shape=jax.ShapeDtypeStruct((m, n), jnp.float32),
    )(x, w)


def _agg(g, srcR, dstR):
    f = g.shape[1]
    fh = f // 2
    g2 = jnp.stack([g[:, :fh], g[:, fh:]])
    p = _sc_agg[f](g2, srcR, dstR).reshape(NC, NP, fh)
    return jnp.concatenate([p[0, :N], p[1, :N]], axis=1)


def kernel(x, edge_index, batch, W1, b1, W2, b2, Wc0, bc0, Wc1, bc1,
           Wf1, bf1, Wf2, bf2, Wf3, bf3):
    src, dst = edge_index[0], edge_index[1]
    act = jax.nn.elu

    pad = E_PAD - E
    src_p = jnp.concatenate([src, jnp.zeros((pad,), jnp.int32)])
    dst_p = jnp.concatenate([dst, jnp.full((pad,), N, jnp.int32)])
    srcR = src_p.reshape(NS, NCHUNK_T, CHUNK)
    dstR = dst_p.reshape(NS, NCHUNK_T, CHUNK)

    cnt = _sc_count(dst_p.reshape(NW, NCHUNK, CHUNK)).reshape(NC, NP, 16)
    deg = cnt[0, :N, 0] + cnt[1, :N, 0] + 1.0
    dinv = jax.lax.rsqrt(deg)

    def conv(h, b):
        g = dinv[:, None] * h
        return dinv[:, None] * (_agg(g, srcR, dstR) + g) + b

    x1 = act(conv(_pallas_matmul(x, W1), b1))
    x3 = act(conv(_pallas_matmul(x1, W2), b2))
    x3 = act(conv(_pallas_matmul(x3, Wc0), bc0))
    x3 = act(conv(_pallas_matmul(x3, Wc1), bc1))

    pooled = jax.ops.segment_max(x3, batch, num_segments=N_GRAPHS,
                                 indices_are_sorted=True)
    pooled = jnp.where(jnp.isfinite(pooled), pooled, 0.0)
    h = act(pooled @ Wf1 + bf1)
    h = act(h @ Wf2 + bf2)
    logits = h @ Wf3 + bf3
    return jax.nn.log_softmax(logits, axis=1)
